# baseline (device time: 183858 ns/iter reference)
import jax
import jax.numpy as jnp
from jax import lax
from jax.experimental import pallas as pl
from jax.experimental.pallas import tpu as pltpu

N_DEV = 8


def kernel(x, w_mat):
    m_per, k = x.shape
    _, n_per = w_mat.shape
    m_glob = m_per * N_DEV

    def body(x_ref, w_ref, out_ref, gather_ref, send_sems, recv_sems):
        my = lax.axis_index("i")
        left = (my - 1) % N_DEV
        right = (my + 1) % N_DEV

        barrier_sem = pltpu.get_barrier_semaphore()
        for nbr in (left, right):
            pl.semaphore_signal(
                barrier_sem, inc=1,
                device_id=(nbr,), device_id_type=pl.DeviceIdType.MESH,
            )
        pl.semaphore_wait(barrier_sem, 2)

        gather_ref[0, :, :] = x_ref[:, :]

        own = jnp.dot(x_ref[:, :], w_ref[:, :],
                      preferred_element_type=jnp.float32)
        out_ref[pl.ds(my * m_per, m_per), :] = jnp.maximum(own, 0.0)

        for h in range(N_DEV - 1):
            rdma = pltpu.make_async_remote_copy(
                src_ref=gather_ref.at[h],
                dst_ref=gather_ref.at[h + 1],
                send_sem=send_sems.at[h],
                recv_sem=recv_sems.at[h],
                device_id=(right,),
                device_id_type=pl.DeviceIdType.MESH,
            )
            rdma.start()
            rdma.wait()

            origin = (my - h - 1) % N_DEV
            blk = jnp.dot(gather_ref[h + 1, :, :], w_ref[:, :],
                          preferred_element_type=jnp.float32)
            out_ref[pl.ds(origin * m_per, m_per), :] = jnp.maximum(blk, 0.0)

    return pl.pallas_call(
        body,
        out_shape=jax.ShapeDtypeStruct((m_glob, n_per), jnp.float32),
        in_specs=[
            pl.BlockSpec(memory_space=pltpu.VMEM),
            pl.BlockSpec(memory_space=pltpu.VMEM),
        ],
        out_specs=pl.BlockSpec(memory_space=pltpu.VMEM),
        scratch_shapes=[
            pltpu.VMEM((N_DEV, m_per, k), jnp.float32),
            pltpu.SemaphoreType.DMA((N_DEV - 1,)),
            pltpu.SemaphoreType.DMA((N_DEV - 1,)),
        ],
        compiler_params=pltpu.CompilerParams(collective_id=0),
    )(x, w_mat)


# device time: 66669 ns/iter; 2.7578x vs baseline; 2.7578x over previous
import jax
import jax.numpy as jnp
from jax import lax
from jax.experimental import pallas as pl
from jax.experimental.pallas import tpu as pltpu

N_DEV = 8

_C_SLICES = ((0, 88), (88, 88), (176, 80))


def kernel(x, w_mat):
    m_per, k = x.shape
    _, n_per = w_mat.shape
    m_glob = m_per * N_DEV

    def body(x_ref, w_ref, out_ref, g_ref, send_sems, recv_sems):
        my = lax.axis_index("i")
        partners = (my ^ 1, my ^ 3, my ^ 4)

        def g_at(origin, row0=None, rows=None):
            if row0 is None:
                return g_ref.at[pl.ds(origin * m_per, m_per), :]
            return g_ref.at[pl.ds(origin * m_per + row0, rows), :]

        def relu_gemm(chunk_ref_rows, origin):
            blk = jnp.dot(chunk_ref_rows, w_ref[:, :],
                          preferred_element_type=jnp.float32)
            out_ref[pl.ds(origin * m_per, m_per), :] = jnp.maximum(blk, 0.0)

        barrier_sem = pltpu.get_barrier_semaphore()
        for p in partners:
            pl.semaphore_signal(
                barrier_sem, inc=1,
                device_id=(p,), device_id_type=pl.DeviceIdType.MESH,
            )
        pl.semaphore_wait(barrier_sem, 3)

        descs = []

        step_a = []
        for l, p in enumerate(partners):
            d = pltpu.make_async_remote_copy(
                src_ref=x_ref.at[:, :],
                dst_ref=g_at(my),
                send_sem=send_sems.at[0, l],
                recv_sem=recv_sems.at[0, l],
                device_id=(p,),
                device_id_type=pl.DeviceIdType.MESH,
            )
            d.start()
            step_a.append(d)
        descs += step_a

        blk = jnp.dot(x_ref[:, :], w_ref[:, :],
                      preferred_element_type=jnp.float32)
        out_ref[pl.ds(my * m_per, m_per), :] = jnp.maximum(blk, 0.0)

        for d in step_a:
            d.wait_recv()

        fwd = (my ^ 3, my ^ 4, my ^ 1)
        step_b = []
        for l, (p, o) in enumerate(zip(partners, fwd)):
            d = pltpu.make_async_remote_copy(
                src_ref=g_at(o),
                dst_ref=g_at(o),
                send_sem=send_sems.at[1, l],
                recv_sem=recv_sems.at[1, l],
                device_id=(p,),
                device_id_type=pl.DeviceIdType.MESH,
            )
            d.start()
            step_b.append(d)
        descs += step_b

        for o in (my ^ 1, my ^ 3, my ^ 4):
            relu_gemm(g_ref[pl.ds(o * m_per, m_per), :], o)

        for d in step_b:
            d.wait_recv()

        opp_src = (my ^ 7, my ^ 5, my ^ 2)
        step_c = []
        for l, (p, o, (r0, nr)) in enumerate(zip(partners, opp_src, _C_SLICES)):
            d = pltpu.make_async_remote_copy(
                src_ref=g_at(o, r0, nr),
                dst_ref=g_at(o, r0, nr),
                send_sem=send_sems.at[2, l],
                recv_sem=recv_sems.at[2, l],
                device_id=(p,),
                device_id_type=pl.DeviceIdType.MESH,
            )
            d.start()
            step_c.append(d)
        descs += step_c

        for o in (my ^ 2, my ^ 7, my ^ 5):
            relu_gemm(g_ref[pl.ds(o * m_per, m_per), :], o)

        for d in step_c:
            d.wait_recv()

        relu_gemm(g_ref[pl.ds((my ^ 6) * m_per, m_per), :], my ^ 6)

        for d in descs:
            d.wait_send()

    return pl.pallas_call(
        body,
        out_shape=jax.ShapeDtypeStruct((m_glob, n_per), jnp.float32),
        in_specs=[
            pl.BlockSpec(memory_space=pltpu.VMEM),
            pl.BlockSpec(memory_space=pltpu.VMEM),
        ],
        out_specs=pl.BlockSpec(memory_space=pltpu.VMEM),
        scratch_shapes=[
            pltpu.VMEM((m_glob, k), jnp.float32),
            pltpu.SemaphoreType.DMA((3, 3)),
            pltpu.SemaphoreType.DMA((3, 3)),
        ],
        compiler_params=pltpu.CompilerParams(collective_id=0),
    )(x, w_mat)


# device time: 65507 ns/iter; 2.8067x vs baseline; 1.0177x over previous
import jax
import jax.numpy as jnp
from jax import lax
from jax.experimental import pallas as pl
from jax.experimental.pallas import tpu as pltpu

N_DEV = 8

_HALVES = ((0, 128), (128, 128))
_C_SLICES = ((0, 88), (88, 88), (176, 80))


def kernel(x, w_mat):
    m_per, k = x.shape
    _, n_per = w_mat.shape
    m_glob = m_per * N_DEV

    def body(x_ref, w_ref, out_ref, g_ref, send_sems, recv_sems):
        my = lax.axis_index("i")
        partners = (my ^ 1, my ^ 3, my ^ 4)

        def g_at(origin, row0=0, rows=m_per):
            return g_ref.at[pl.ds(origin * m_per + row0, rows), :]

        def relu_gemm(origin):
            blk = jnp.dot(g_ref[pl.ds(origin * m_per, m_per), :], w_ref[:, :],
                          preferred_element_type=jnp.float32)
            out_ref[pl.ds(origin * m_per, m_per), :] = jnp.maximum(blk, 0.0)

        def copy(sem_idx, link, src_ref, dst_ref):
            return pltpu.make_async_remote_copy(
                src_ref=src_ref,
                dst_ref=dst_ref,
                send_sem=send_sems.at[sem_idx],
                recv_sem=recv_sems.at[sem_idx],
                device_id=(partners[link],),
                device_id_type=pl.DeviceIdType.MESH,
            )

        barrier_sem = pltpu.get_barrier_semaphore()
        for p in partners:
            pl.semaphore_signal(
                barrier_sem, inc=1,
                device_id=(p,), device_id_type=pl.DeviceIdType.MESH,
            )
        pl.semaphore_wait(barrier_sem, 3)

        A = {}
        for h, (r0, nr) in enumerate(_HALVES):
            for l in range(3):
                A[l, h] = copy(h * 3 + l, l,
                               x_ref.at[pl.ds(r0, nr), :], g_at(my, r0, nr))
                A[l, h].start()

        blk = jnp.dot(x_ref[:, :], w_ref[:, :],
                      preferred_element_type=jnp.float32)
        out_ref[pl.ds(my * m_per, m_per), :] = jnp.maximum(blk, 0.0)

        fwd = (my ^ 3, my ^ 4, my ^ 1)
        src_link = (1, 2, 0)
        B = {}
        for h, (r0, nr) in enumerate(_HALVES):
            for l in range(3):
                A[src_link[l], h].wait_recv()
                B[l, h] = copy(6 + h * 3 + l, l,
                               g_at(fwd[l], r0, nr), g_at(fwd[l], r0, nr))
                B[l, h].start()

        for o in (my ^ 1, my ^ 3, my ^ 4):
            relu_gemm(o)

        opp_src = (my ^ 7, my ^ 5, my ^ 2)
        C = {}

        def start_c(l):
            r0, nr = _C_SLICES[l]
            C[l] = copy(12 + l, l, g_at(opp_src[l], r0, nr),
                        g_at(opp_src[l], r0, nr))
            C[l].start()

        B[1, 0].wait_recv()
        start_c(0)
        B[2, 0].wait_recv()
        B[0, 0].wait_recv()
        B[2, 1].wait_recv()
        start_c(1)
        B[0, 1].wait_recv()
        start_c(2)
        B[1, 1].wait_recv()

        for o in (my ^ 2, my ^ 7, my ^ 5):
            relu_gemm(o)

        for l in range(3):
            C[l].wait_recv()

        relu_gemm(my ^ 6)

        for d in list(A.values()) + list(B.values()) + list(C.values()):
            d.wait_send()

    return pl.pallas_call(
        body,
        out_shape=jax.ShapeDtypeStruct((m_glob, n_per), jnp.float32),
        in_specs=[
            pl.BlockSpec(memory_space=pltpu.VMEM),
            pl.BlockSpec(memory_space=pltpu.VMEM),
        ],
        out_specs=pl.BlockSpec(memory_space=pltpu.VMEM),
        scratch_shapes=[
            pltpu.VMEM((m_glob, k), jnp.float32),
            pltpu.SemaphoreType.DMA((15,)),
            pltpu.SemaphoreType.DMA((15,)),
        ],
        compiler_params=pltpu.CompilerParams(collective_id=0),
    )(x, w_mat)
